# baseline (speedup 1.0000x reference)
import jax
import jax.numpy as jnp
from jax import lax
from jax.experimental import pallas as pl
from jax.experimental.pallas import tpu as pltpu
from jax.experimental.pallas import tpu_sc as plsc

_DELTA_VAR = 0.1
_DELTA_DIST = 0.3
_B = 2
_C = 128
_D = 32
_P = 4096
_E = 512
_NI = 4
_NS = 16
_TP = _P // 4
_L = 16
_AR = _D + 1


def _splat_f(x):
    return lax.broadcast_in_dim(x, (_L,), ())


def _rsqrt_newton(n):
    i = plsc.bitcast(n, jnp.int32)
    i = jnp.int32(0x5F3759DF) - lax.shift_right_logical(i, 1)
    y = plsc.bitcast(i, jnp.float32)
    for _ in range(3):
        y = y * (1.5 - 0.5 * n * y * y)
    return y


def _sc_body(emb_hbm, seg_hbm, val_hbm, edges_hbm, w_hbm, misc_hbm, zero_hbm,
             out_hbm,
             seg_v, emb_v, acc_v, cmb_v, mn_v, wq_v, mult_v, val_v, edges_v,
             we_v, misc_v, mc_v, mc2_v, part_v, partall_v, out_v, idx_v,
             sems,
             shared_acc, shared_mc, shared_part):
    f32 = jnp.float32
    i32 = jnp.int32
    s = lax.axis_index("s")
    j = s // 4
    q = s % 4
    batch = j % 2
    iota = lax.iota(i32, _L)
    zeros = jnp.zeros((_L,), f32)
    ones = jnp.ones((_L,), f32)
    zl = jnp.zeros((_L,), i32)

    cps = [
        pltpu.make_async_copy(seg_hbm.at[j, pl.ds(q * _TP, _TP)], seg_v,
                              sems.at[0]),
        pltpu.make_async_copy(emb_hbm.at[j, :, pl.ds(q * _TP, _TP)], emb_v,
                              sems.at[1]),
        pltpu.make_async_copy(val_hbm.at[batch], val_v, sems.at[2]),
        pltpu.make_async_copy(edges_hbm.at[batch], edges_v, sems.at[3]),
        pltpu.make_async_copy(w_hbm.at[batch], we_v, sems.at[4]),
        pltpu.make_async_copy(misc_hbm, misc_v, sems.at[5]),
        pltpu.make_async_copy(zero_hbm, acc_v, sems.at[6]),
    ]
    for cp in cps:
        cp.start()

    idx_v[pl.ds(0, _L)] = iota + j * _AR
    idx_v[pl.ds(_L, _L)] = iota + (j * _AR + _L)
    idx_v[pl.ds(_AR - _L, _L)] = iota + (j * _AR + _AR - _L)

    for cp in cps:
        cp.wait()

    @pl.when(q == 0)
    def _zero_zone():
        pltpu.sync_copy(acc_v, shared_acc.at[pl.ds(j * _AR, _AR)])
    plsc.subcore_barrier()

    row_cnt = jnp.full((_L,), _D, i32)

    @plsc.parallel_loop(0, _TP // _L, step=1, unroll=2)
    def scatter_group(grp):
        base = grp * _L
        s_l = seg_v[pl.ds(base, _L)]
        plsc.addupdate_scatter(acc_v, [row_cnt, s_l], ones)
        for d in range(_D):
            e = emb_v[d, pl.ds(base, _L)]
            plsc.addupdate_scatter(acc_v, [jnp.full((_L,), d, i32), s_l], e)

    pltpu.sync_copy(acc_v, shared_acc.at[idx_v], add=True)
    plsc.subcore_barrier()
    pltpu.sync_copy(shared_acc.at[pl.ds(j * _AR, _AR)], cmb_v)

    for g in range(_C // _L):
        mult_v[0, pl.ds(g * _L, _L)] = zeros
    for g in range(_C // _L):
        vl = val_v[pl.ds(g * _L, _L)]
        plsc.addupdate_scatter(mult_v, [zl, vl], ones)

    def norm_group(g, _):
        cnt = cmb_v[_D, pl.ds(g * _L, _L)]
        inv_cnt = 1.0 / cnt
        ms = [cmb_v[d, pl.ds(g * _L, _L)] * inv_cnt for d in range(_D)]
        n0 = zeros
        n1 = zeros
        for d in range(16):
            n0 = n0 + ms[d] * ms[d]
            n1 = n1 + ms[d + 16] * ms[d + 16]
        y0 = _rsqrt_newton(n0)
        y1 = _rsqrt_newton(n1)
        for d in range(16):
            mn_v[d, pl.ds(g * _L, _L)] = ms[d] * y0 + 1e-10
            mn_v[d + 16, pl.ds(g * _L, _L)] = ms[d + 16] * y1 + 1e-10
        wq_v[0, pl.ds(g * _L, _L)] = mult_v[0, pl.ds(g * _L, _L)] * inv_cnt
        return 0
    lax.fori_loop(0, _C // _L, norm_group, 0)

    @plsc.parallel_loop(0, _TP // _L, step=1, unroll=2, carry=zeros)
    def intra_group(grp, acc):
        base = grp * _L
        s_l = seg_v[pl.ds(base, _L)]
        dot = zeros
        for d in range(_D):
            g = plsc.load_gather(mn_v, [jnp.full((_L,), d, i32), s_l])
            e = emb_v[d, pl.ds(base, _L)]
            dot = dot + g * e
        wl = plsc.load_gather(wq_v, [zl, s_l])
        return acc + wl * jnp.maximum((2.0 - dot) * 0.5 - _DELTA_VAR, 0.0)
    iacc = intra_group
    part_v[...] = iacc * (1.0 / _C)

    def mc_group(g, _):
        vl = val_v[pl.ds(g * _L, _L)]
        for d in range(_D):
            mc_v[d, pl.ds(g * _L, _L)] = plsc.load_gather(mn_v, [zl + d, vl])
        return 0
    lax.fori_loop(0, _C // _L, mc_group, 0)

    @pl.when(q == 0)
    def _publish_mc():
        pltpu.sync_copy(mc_v, shared_mc.at[j])
    plsc.subcore_barrier()

    def edge_group(grp, acc):
        base = grp * _L
        e0 = edges_v[0, pl.ds(base, _L)]
        e1 = edges_v[1, pl.ds(base, _L)]
        sm = zeros
        for d in range(16):
            dl = jnp.full((_L,), d, i32)
            a = plsc.load_gather(mc_v, [dl, e0])
            b = plsc.load_gather(mc_v, [dl, e1])
            sm = sm + a * b
        inter = (1.0 - sm) * we_v[pl.ds(base, _L)]
        return acc + jnp.maximum(_DELTA_DIST - inter, 0.0)
    ngrp = _E // _L // 4
    eacc = lax.fori_loop(q * ngrp, (q + 1) * ngrp, edge_group, zeros)
    part_v[...] = part_v[...] + eacc * (1.0 / _E)

    @pl.when(jnp.logical_and(q == 1, j < 2))
    def _rd():
        pltpu.sync_copy(shared_mc.at[j], mc_v)
        pltpu.sync_copy(shared_mc.at[j + 2], mc2_v)

        def rd_acc(off):
            def body(d, acc):
                return acc + (mc_v[d, pl.ds(off, _L)]
                              * mc2_v[d, pl.ds(off, _L)])
            return lax.fori_loop(0, _D, body, zeros)
        s0 = rd_acc(0)
        s1 = rd_acc(16)
        rd1 = 1.0 - _splat_f(jnp.sum(s0)) * (1.0 / _D)
        rd2 = 1.0 - _splat_f(jnp.sum(s1)) * (1.0 / _D)
        ang = misc_v[...]
        term = (jnp.maximum(rd1 - _DELTA_VAR, 0.0)
                + jnp.maximum(_DELTA_DIST - rd2, 0.0) * ang) * (1.0 / _L)
        part_v[...] = part_v[...] + term

    pltpu.sync_copy(part_v, shared_part.at[s])
    plsc.subcore_barrier()

    @pl.when(s == 0)
    def _final():
        pltpu.sync_copy(shared_part, partall_v)
        tot = zeros
        for t in range(_NS):
            tot = tot + partall_v[t, pl.ds(0, _L)]
        out_v[...] = _splat_f(jnp.sum(tot))
        pltpu.sync_copy(out_v, out_hbm)


@jax.jit
def _run(embeddings, sp_seg, rot_sp, edges, weights, val_sp, rotation_angle):
    f32 = jnp.float32
    emb = embeddings.reshape(2 * _B, _D, _P)
    seg = jnp.concatenate([sp_seg.reshape(_B, _P),
                           rot_sp.reshape(_B, _P)], axis=0)
    misc = jnp.full((_L,), rotation_angle, f32)
    zero = jnp.zeros((_AR, _C), f32)

    mesh = plsc.VectorSubcoreMesh(core_axis_name="c", subcore_axis_name="s",
                                  num_cores=1)
    sc_call = pl.kernel(
        _sc_body,
        out_type=jax.ShapeDtypeStruct((_L,), f32),
        mesh=mesh,
        scratch_types=[
            pltpu.VMEM((_TP,), jnp.int32),
            pltpu.VMEM((_D, _TP), f32),
            pltpu.VMEM((_AR, _C), f32),
            pltpu.VMEM((_AR, _C), f32),
            pltpu.VMEM((_D, _C), f32),
            pltpu.VMEM((1, _C), f32),
            pltpu.VMEM((1, _C), f32),
            pltpu.VMEM((_C,), jnp.int32),
            pltpu.VMEM((2, _E), jnp.int32),
            pltpu.VMEM((_E,), f32),
            pltpu.VMEM((_L,), f32),
            pltpu.VMEM((_D, _C), f32),
            pltpu.VMEM((_D, _C), f32),
            pltpu.VMEM((_L,), f32),
            pltpu.VMEM((_NS, _L), f32),
            pltpu.VMEM((_L,), f32),
            pltpu.VMEM((_AR,), jnp.int32),
            pltpu.SemaphoreType.DMA((7,)),
            pltpu.VMEM_SHARED((_NI * _AR, _C), f32),
            pltpu.VMEM_SHARED((_NI, _D, _C), f32),
            pltpu.VMEM_SHARED((_NS, _L), f32),
        ],
        compiler_params=pltpu.CompilerParams(use_tc_tiling_on_sc=False,
                                             needs_layout_passes=False),
    )
    out = sc_call(emb, seg, val_sp, edges, weights, misc, zero)
    return out[0]


def kernel(embeddings, sp_seg, rot_sp, edges, weights, val_sp, rotation_angle):
    return _run(embeddings, sp_seg, rot_sp, edges, weights, val_sp,
                jnp.asarray(rotation_angle, jnp.float32))


# speedup vs baseline: 1.0138x; 1.0138x over previous
import jax
import jax.numpy as jnp
from jax import lax
from jax.experimental import pallas as pl
from jax.experimental.pallas import tpu as pltpu
from jax.experimental.pallas import tpu_sc as plsc

_DELTA_VAR = 0.1
_DELTA_DIST = 0.3
_B = 2
_C = 128
_D = 32
_P = 4096
_E = 512
_NI = 4
_NS = 16
_TP = _P // 4
_L = 16
_AR = _D + 1


def _splat_f(x):
    return lax.broadcast_in_dim(x, (_L,), ())


def _rsqrt_newton(n):
    i = plsc.bitcast(n, jnp.int32)
    i = jnp.int32(0x5F3759DF) - lax.shift_right_logical(i, 1)
    y = plsc.bitcast(i, jnp.float32)
    for _ in range(3):
        y = y * (1.5 - 0.5 * n * y * y)
    return y


def _sc_body(emb_hbm, seg_hbm, val_hbm, edges_hbm, w_hbm, misc_hbm, zero_hbm,
             out_hbm,
             seg_v, emb_v, acc_v, cmb_v, mn_v, wq_v, mult_v, val_v, edges_v,
             we_v, misc_v, mc_v, mc2_v, part_v, partall_v, out_v, idx_v,
             sems,
             shared_acc, shared_mc, shared_part):
    f32 = jnp.float32
    i32 = jnp.int32
    s = lax.axis_index("s")
    j = s // 4
    q = s % 4
    batch = j % 2
    iota = lax.iota(i32, _L)
    zeros = jnp.zeros((_L,), f32)
    ones = jnp.ones((_L,), f32)
    zl = jnp.zeros((_L,), i32)

    cps = [
        pltpu.make_async_copy(seg_hbm.at[j, pl.ds(q * _TP, _TP)], seg_v,
                              sems.at[0]),
        pltpu.make_async_copy(emb_hbm.at[j, :, pl.ds(q * _TP, _TP)], emb_v,
                              sems.at[1]),
        pltpu.make_async_copy(val_hbm.at[batch], val_v, sems.at[2]),
        pltpu.make_async_copy(edges_hbm.at[batch], edges_v, sems.at[3]),
        pltpu.make_async_copy(w_hbm.at[batch], we_v, sems.at[4]),
        pltpu.make_async_copy(misc_hbm, misc_v, sems.at[5]),
        pltpu.make_async_copy(zero_hbm, acc_v, sems.at[6]),
    ]
    for cp in cps:
        cp.start()

    idx_v[pl.ds(0, _L)] = iota + j * _AR
    idx_v[pl.ds(_L, _L)] = iota + (j * _AR + _L)
    idx_v[pl.ds(_AR - _L, _L)] = iota + (j * _AR + _AR - _L)

    for cp in cps:
        cp.wait()

    @pl.when(q == 0)
    def _zero_zone():
        pltpu.sync_copy(acc_v, shared_acc.at[pl.ds(j * _AR, _AR)])
    plsc.subcore_barrier()

    row_cnt = jnp.full((_L,), _D, i32)

    @plsc.parallel_loop(0, _TP // _L, step=1, unroll=2)
    def scatter_group(grp):
        base = grp * _L
        s_l = seg_v[pl.ds(base, _L)]
        plsc.addupdate_scatter(acc_v.at[_D], [s_l], ones)
        for d in range(_D):
            e = emb_v[d, pl.ds(base, _L)]
            plsc.addupdate_scatter(acc_v.at[d], [s_l], e)

    pltpu.sync_copy(acc_v, shared_acc.at[idx_v], add=True)
    plsc.subcore_barrier()
    pltpu.sync_copy(shared_acc.at[pl.ds(j * _AR, _AR)], cmb_v)

    for g in range(_C // _L):
        mult_v[0, pl.ds(g * _L, _L)] = zeros
    for g in range(_C // _L):
        vl = val_v[pl.ds(g * _L, _L)]
        plsc.addupdate_scatter(mult_v, [zl, vl], ones)

    def norm_group(g, _):
        cnt = cmb_v[_D, pl.ds(g * _L, _L)]
        inv_cnt = 1.0 / cnt
        ms = [cmb_v[d, pl.ds(g * _L, _L)] * inv_cnt for d in range(_D)]
        n0 = zeros
        n1 = zeros
        for d in range(16):
            n0 = n0 + ms[d] * ms[d]
            n1 = n1 + ms[d + 16] * ms[d + 16]
        y0 = _rsqrt_newton(n0)
        y1 = _rsqrt_newton(n1)
        for d in range(16):
            mn_v[d, pl.ds(g * _L, _L)] = ms[d] * y0 + 1e-10
            mn_v[d + 16, pl.ds(g * _L, _L)] = ms[d + 16] * y1 + 1e-10
        wq_v[0, pl.ds(g * _L, _L)] = mult_v[0, pl.ds(g * _L, _L)] * inv_cnt
        return 0
    lax.fori_loop(0, _C // _L, norm_group, 0)

    @plsc.parallel_loop(0, _TP // _L, step=1, unroll=2, carry=zeros)
    def intra_group(grp, acc):
        base = grp * _L
        s_l = seg_v[pl.ds(base, _L)]
        dot = zeros
        for d in range(_D):
            g = plsc.load_gather(mn_v.at[d], [s_l])
            e = emb_v[d, pl.ds(base, _L)]
            dot = dot + g * e
        wl = plsc.load_gather(wq_v.at[0], [s_l])
        return acc + wl * jnp.maximum((2.0 - dot) * 0.5 - _DELTA_VAR, 0.0)
    iacc = intra_group
    part_v[...] = iacc * (1.0 / _C)

    def mc_group(g, _):
        vl = val_v[pl.ds(g * _L, _L)]
        for d in range(_D):
            mc_v[d, pl.ds(g * _L, _L)] = plsc.load_gather(mn_v, [zl + d, vl])
        return 0
    lax.fori_loop(0, _C // _L, mc_group, 0)

    @pl.when(q == 0)
    def _publish_mc():
        pltpu.sync_copy(mc_v, shared_mc.at[j])
    plsc.subcore_barrier()

    def edge_group(grp, acc):
        base = grp * _L
        e0 = edges_v[0, pl.ds(base, _L)]
        e1 = edges_v[1, pl.ds(base, _L)]
        sm = zeros
        for d in range(16):
            dl = jnp.full((_L,), d, i32)
            a = plsc.load_gather(mc_v, [dl, e0])
            b = plsc.load_gather(mc_v, [dl, e1])
            sm = sm + a * b
        inter = (1.0 - sm) * we_v[pl.ds(base, _L)]
        return acc + jnp.maximum(_DELTA_DIST - inter, 0.0)
    ngrp = _E // _L // 4
    eacc = lax.fori_loop(q * ngrp, (q + 1) * ngrp, edge_group, zeros)
    part_v[...] = part_v[...] + eacc * (1.0 / _E)

    @pl.when(jnp.logical_and(q == 1, j < 2))
    def _rd():
        pltpu.sync_copy(shared_mc.at[j], mc_v)
        pltpu.sync_copy(shared_mc.at[j + 2], mc2_v)

        def rd_acc(off):
            def body(d, acc):
                return acc + (mc_v[d, pl.ds(off, _L)]
                              * mc2_v[d, pl.ds(off, _L)])
            return lax.fori_loop(0, _D, body, zeros)
        s0 = rd_acc(0)
        s1 = rd_acc(16)
        rd1 = 1.0 - _splat_f(jnp.sum(s0)) * (1.0 / _D)
        rd2 = 1.0 - _splat_f(jnp.sum(s1)) * (1.0 / _D)
        ang = misc_v[...]
        term = (jnp.maximum(rd1 - _DELTA_VAR, 0.0)
                + jnp.maximum(_DELTA_DIST - rd2, 0.0) * ang) * (1.0 / _L)
        part_v[...] = part_v[...] + term

    pltpu.sync_copy(part_v, shared_part.at[s])
    plsc.subcore_barrier()

    @pl.when(s == 0)
    def _final():
        pltpu.sync_copy(shared_part, partall_v)
        tot = zeros
        for t in range(_NS):
            tot = tot + partall_v[t, pl.ds(0, _L)]
        out_v[...] = _splat_f(jnp.sum(tot))
        pltpu.sync_copy(out_v, out_hbm)


@jax.jit
def _run(embeddings, sp_seg, rot_sp, edges, weights, val_sp, rotation_angle):
    f32 = jnp.float32
    emb = embeddings.reshape(2 * _B, _D, _P)
    seg = jnp.concatenate([sp_seg.reshape(_B, _P),
                           rot_sp.reshape(_B, _P)], axis=0)
    misc = jnp.full((_L,), rotation_angle, f32)
    zero = jnp.zeros((_AR, _C), f32)

    mesh = plsc.VectorSubcoreMesh(core_axis_name="c", subcore_axis_name="s",
                                  num_cores=1)
    sc_call = pl.kernel(
        _sc_body,
        out_type=jax.ShapeDtypeStruct((_L,), f32),
        mesh=mesh,
        scratch_types=[
            pltpu.VMEM((_TP,), jnp.int32),
            pltpu.VMEM((_D, _TP), f32),
            pltpu.VMEM((_AR, _C), f32),
            pltpu.VMEM((_AR, _C), f32),
            pltpu.VMEM((_D, _C), f32),
            pltpu.VMEM((1, _C), f32),
            pltpu.VMEM((1, _C), f32),
            pltpu.VMEM((_C,), jnp.int32),
            pltpu.VMEM((2, _E), jnp.int32),
            pltpu.VMEM((_E,), f32),
            pltpu.VMEM((_L,), f32),
            pltpu.VMEM((_D, _C), f32),
            pltpu.VMEM((_D, _C), f32),
            pltpu.VMEM((_L,), f32),
            pltpu.VMEM((_NS, _L), f32),
            pltpu.VMEM((_L,), f32),
            pltpu.VMEM((_AR,), jnp.int32),
            pltpu.SemaphoreType.DMA((7,)),
            pltpu.VMEM_SHARED((_NI * _AR, _C), f32),
            pltpu.VMEM_SHARED((_NI, _D, _C), f32),
            pltpu.VMEM_SHARED((_NS, _L), f32),
        ],
        compiler_params=pltpu.CompilerParams(use_tc_tiling_on_sc=False,
                                             needs_layout_passes=False),
    )
    out = sc_call(emb, seg, val_sp, edges, weights, misc, zero)
    return out[0]


def kernel(embeddings, sp_seg, rot_sp, edges, weights, val_sp, rotation_angle):
    return _run(embeddings, sp_seg, rot_sp, edges, weights, val_sp,
                jnp.asarray(rotation_angle, jnp.float32))


# speedup vs baseline: 1.0222x; 1.0083x over previous
import jax
import jax.numpy as jnp
from jax import lax
from jax.experimental import pallas as pl
from jax.experimental.pallas import tpu as pltpu
from jax.experimental.pallas import tpu_sc as plsc

_DELTA_VAR = 0.1
_DELTA_DIST = 0.3
_B = 2
_C = 128
_D = 32
_P = 4096
_E = 512
_NI = 4
_NS = 16
_TP = _P // 4
_L = 16
_AR = _D + 1


def _splat_f(x):
    return lax.broadcast_in_dim(x, (_L,), ())


def _rsqrt_newton(n):
    i = plsc.bitcast(n, jnp.int32)
    i = jnp.int32(0x5F3759DF) - lax.shift_right_logical(i, 1)
    y = plsc.bitcast(i, jnp.float32)
    for _ in range(3):
        y = y * (1.5 - 0.5 * n * y * y)
    return y


def _sc_body(emb_hbm, seg_hbm, val_hbm, edges_hbm, w_hbm, misc_hbm, zero_hbm,
             out_hbm,
             seg_v, emb_v, acc_v, cmb_v, mn_v, wq_v, mult_v, val_v, edges_v,
             we_v, misc_v, mc_v, mc2_v, part_v, partall_v, out_v, idx_v,
             sems,
             shared_acc, shared_mc, shared_part):
    f32 = jnp.float32
    i32 = jnp.int32
    s = lax.axis_index("s")
    j = s // 4
    q = s % 4
    batch = j % 2
    iota = lax.iota(i32, _L)
    zeros = jnp.zeros((_L,), f32)
    ones = jnp.ones((_L,), f32)
    zl = jnp.zeros((_L,), i32)

    cps = [
        pltpu.make_async_copy(seg_hbm.at[j, pl.ds(q * _TP, _TP)], seg_v,
                              sems.at[0]),
        pltpu.make_async_copy(emb_hbm.at[j, :, pl.ds(q * _TP, _TP)], emb_v,
                              sems.at[1]),
        pltpu.make_async_copy(val_hbm.at[batch], val_v, sems.at[2]),
        pltpu.make_async_copy(edges_hbm.at[batch], edges_v, sems.at[3]),
        pltpu.make_async_copy(w_hbm.at[batch], we_v, sems.at[4]),
        pltpu.make_async_copy(misc_hbm, misc_v, sems.at[5]),
        pltpu.make_async_copy(zero_hbm, acc_v, sems.at[6]),
    ]
    for cp in cps:
        cp.start()

    idx_v[pl.ds(0, _L)] = iota + j * _AR
    idx_v[pl.ds(_L, _L)] = iota + (j * _AR + _L)
    idx_v[pl.ds(_AR - _L, _L)] = iota + (j * _AR + _AR - _L)

    for cp in cps:
        cp.wait()

    @pl.when(q == 0)
    def _zero_zone():
        pltpu.sync_copy(acc_v, shared_acc.at[pl.ds(j * _AR, _AR)])
    plsc.subcore_barrier()

    row_cnt = jnp.full((_L,), _D, i32)

    @plsc.parallel_loop(0, _TP // _L, step=1, unroll=2)
    def scatter_group(grp):
        base = grp * _L
        s_l = seg_v[pl.ds(base, _L)]
        plsc.addupdate_scatter(acc_v.at[_D], [s_l], ones)
        for d in range(_D):
            e = emb_v[d, pl.ds(base, _L)]
            plsc.addupdate_scatter(acc_v.at[d], [s_l], e)

    pltpu.sync_copy(acc_v, shared_acc.at[idx_v], add=True)
    plsc.subcore_barrier()
    pltpu.sync_copy(shared_acc.at[pl.ds(j * _AR, _AR)], cmb_v)

    for g in range(_C // _L):
        mult_v[0, pl.ds(g * _L, _L)] = zeros
    for g in range(_C // _L):
        vl = val_v[pl.ds(g * _L, _L)]
        plsc.addupdate_scatter(mult_v.at[0], [vl], ones)

    def norm_group(g, _):
        cnt = cmb_v[_D, pl.ds(g * _L, _L)]
        inv_cnt = 1.0 / cnt
        ms = [cmb_v[d, pl.ds(g * _L, _L)] * inv_cnt for d in range(_D)]
        n0 = zeros
        n1 = zeros
        for d in range(16):
            n0 = n0 + ms[d] * ms[d]
            n1 = n1 + ms[d + 16] * ms[d + 16]
        y0 = _rsqrt_newton(n0)
        y1 = _rsqrt_newton(n1)
        for d in range(16):
            mn_v[d, pl.ds(g * _L, _L)] = ms[d] * y0 + 1e-10
            mn_v[d + 16, pl.ds(g * _L, _L)] = ms[d + 16] * y1 + 1e-10
        wq_v[0, pl.ds(g * _L, _L)] = mult_v[0, pl.ds(g * _L, _L)] * inv_cnt
        return 0
    lax.fori_loop(0, _C // _L, norm_group, 0)

    @plsc.parallel_loop(0, _TP // _L, step=1, unroll=2, carry=zeros)
    def intra_group(grp, acc):
        base = grp * _L
        s_l = seg_v[pl.ds(base, _L)]
        dot = zeros
        for d in range(_D):
            g = plsc.load_gather(mn_v.at[d], [s_l])
            e = emb_v[d, pl.ds(base, _L)]
            dot = dot + g * e
        wl = plsc.load_gather(wq_v.at[0], [s_l])
        return acc + wl * jnp.maximum((2.0 - dot) * 0.5 - _DELTA_VAR, 0.0)
    iacc = intra_group
    part_v[...] = iacc * (1.0 / _C)

    def mc_group(g, _):
        vl = val_v[pl.ds(g * _L, _L)]
        for d in range(_D):
            mc_v[d, pl.ds(g * _L, _L)] = plsc.load_gather(mn_v.at[d], [vl])
        return 0
    lax.fori_loop(0, _C // _L, mc_group, 0)

    @pl.when(q == 0)
    def _publish_mc():
        pltpu.sync_copy(mc_v, shared_mc.at[j])
    plsc.subcore_barrier()

    def edge_group(grp, acc):
        base = grp * _L
        e0 = edges_v[0, pl.ds(base, _L)]
        e1 = edges_v[1, pl.ds(base, _L)]
        sm = zeros
        for d in range(16):
            a = plsc.load_gather(mc_v.at[d], [e0])
            b = plsc.load_gather(mc_v.at[d], [e1])
            sm = sm + a * b
        inter = (1.0 - sm) * we_v[pl.ds(base, _L)]
        return acc + jnp.maximum(_DELTA_DIST - inter, 0.0)
    ngrp = _E // _L // 4
    eacc = lax.fori_loop(q * ngrp, (q + 1) * ngrp, edge_group, zeros)
    part_v[...] = part_v[...] + eacc * (1.0 / _E)

    @pl.when(jnp.logical_and(q == 1, j < 2))
    def _rd():
        pltpu.sync_copy(shared_mc.at[j], mc_v)
        pltpu.sync_copy(shared_mc.at[j + 2], mc2_v)

        def rd_acc(off):
            def body(d, acc):
                return acc + (mc_v[d, pl.ds(off, _L)]
                              * mc2_v[d, pl.ds(off, _L)])
            return lax.fori_loop(0, _D, body, zeros)
        s0 = rd_acc(0)
        s1 = rd_acc(16)
        rd1 = 1.0 - _splat_f(jnp.sum(s0)) * (1.0 / _D)
        rd2 = 1.0 - _splat_f(jnp.sum(s1)) * (1.0 / _D)
        ang = misc_v[...]
        term = (jnp.maximum(rd1 - _DELTA_VAR, 0.0)
                + jnp.maximum(_DELTA_DIST - rd2, 0.0) * ang) * (1.0 / _L)
        part_v[...] = part_v[...] + term

    pltpu.sync_copy(part_v, shared_part.at[s])
    plsc.subcore_barrier()

    @pl.when(s == 0)
    def _final():
        pltpu.sync_copy(shared_part, partall_v)
        tot = zeros
        for t in range(_NS):
            tot = tot + partall_v[t, pl.ds(0, _L)]
        out_v[...] = _splat_f(jnp.sum(tot))
        pltpu.sync_copy(out_v, out_hbm)


@jax.jit
def _run(embeddings, sp_seg, rot_sp, edges, weights, val_sp, rotation_angle):
    f32 = jnp.float32
    emb = embeddings.reshape(2 * _B, _D, _P)
    seg = jnp.concatenate([sp_seg.reshape(_B, _P),
                           rot_sp.reshape(_B, _P)], axis=0)
    misc = jnp.full((_L,), rotation_angle, f32)
    zero = jnp.zeros((_AR, _C), f32)

    mesh = plsc.VectorSubcoreMesh(core_axis_name="c", subcore_axis_name="s",
                                  num_cores=1)
    sc_call = pl.kernel(
        _sc_body,
        out_type=jax.ShapeDtypeStruct((_L,), f32),
        mesh=mesh,
        scratch_types=[
            pltpu.VMEM((_TP,), jnp.int32),
            pltpu.VMEM((_D, _TP), f32),
            pltpu.VMEM((_AR, _C), f32),
            pltpu.VMEM((_AR, _C), f32),
            pltpu.VMEM((_D, _C), f32),
            pltpu.VMEM((1, _C), f32),
            pltpu.VMEM((1, _C), f32),
            pltpu.VMEM((_C,), jnp.int32),
            pltpu.VMEM((2, _E), jnp.int32),
            pltpu.VMEM((_E,), f32),
            pltpu.VMEM((_L,), f32),
            pltpu.VMEM((_D, _C), f32),
            pltpu.VMEM((_D, _C), f32),
            pltpu.VMEM((_L,), f32),
            pltpu.VMEM((_NS, _L), f32),
            pltpu.VMEM((_L,), f32),
            pltpu.VMEM((_AR,), jnp.int32),
            pltpu.SemaphoreType.DMA((7,)),
            pltpu.VMEM_SHARED((_NI * _AR, _C), f32),
            pltpu.VMEM_SHARED((_NI, _D, _C), f32),
            pltpu.VMEM_SHARED((_NS, _L), f32),
        ],
        compiler_params=pltpu.CompilerParams(use_tc_tiling_on_sc=False,
                                             needs_layout_passes=False),
    )
    out = sc_call(emb, seg, val_sp, edges, weights, misc, zero)
    return out[0]


def kernel(embeddings, sp_seg, rot_sp, edges, weights, val_sp, rotation_angle):
    return _run(embeddings, sp_seg, rot_sp, edges, weights, val_sp,
                jnp.asarray(rotation_angle, jnp.float32))
